# baseline (speedup 1.0000x reference)
import jax
import jax.numpy as jnp
from jax.experimental import pallas as pl
from jax.experimental.pallas import tpu as pltpu

B = 256
D_MODEL = 1024
D_KEY = 64
D_VALUE = 64
DKV = D_KEY * D_VALUE
H = 16
BANK = 4
TOPK = 2
P = H * D_KEY
DT = 512
NT = P // DT
NA = NT * BANK
BB = 16
NCH = B // BB
PRE = 2

f32 = jnp.float32
bf16 = jnp.bfloat16


def _body(x_ref, idx_ref, probs_ref, n2_ref, ndt_ref, md2_ref,
          sel_mat_ref, tile_mat_ref,
          wk_ref, bk_ref, wv_ref, bv_ref, ww_ref, bw_ref, we_ref, be_ref,
          m_hbm, m_auto, om_hbm, on_ref,
          g_ref, mring, obuf, in_sems, out_sems):
    s = pl.program_id(0)

    def in_copy(c):
        return pltpu.make_async_copy(
            m_hbm.at[pl.ds(c * BB, BB)], mring.at[c], in_sems.at[c])

    @pl.when(s == 0)
    def _():
        for c in range(PRE):
            in_copy(c).start()

    @pl.when(s < NA)
    def _():
        t = s // BANK
        n = s % BANK
        t_off = pl.multiple_of(t * DT, DT)
        sn = jnp.zeros((B, 1), f32)
        for k in range(TOPK):
            sn = sn + jnp.where(idx_ref[:, k:k + 1] == n,
                                probs_ref[:, k:k + 1], 0.0)
        x = x_ref[...].astype(bf16)
        for p, (w_ref, b_ref) in enumerate(
                ((wk_ref, bk_ref), (wv_ref, bv_ref),
                 (ww_ref, bw_ref), (we_ref, be_ref))):
            y = jnp.dot(x, w_ref[0].astype(bf16),
                        preferred_element_type=f32)
            bias = b_ref[n, 0:1, pl.ds(t_off, DT)]
            contrib = sn * (y + bias)
            col = pl.ds(p * P + t_off, DT)

            @pl.when(n == 0)
            def _(col=col, contrib=contrib):
                g_ref[:, col] = contrib

            @pl.when(n != 0)
            def _(col=col, contrib=contrib):
                g_ref[:, col] = g_ref[:, col] + contrib

    @pl.when(s >= NA)
    def _():
        hp = s - NA
        rows = pl.ds(pl.multiple_of(hp * BB, BB), BB)

        @pl.when(hp < PRE)
        def _():
            in_copy(hp).wait()

        sel = sel_mat_ref[...]
        til = tile_mat_ref[...]
        md2 = md2_ref[...]

        def update(src_ref, dst_ref):
            for h in range(H):
                c0 = h * D_KEY
                aw = jax.nn.sigmoid(g_ref[rows, 2 * P + c0:2 * P + c0 + D_KEY])
                ae = jax.nn.sigmoid(g_ref[rows, 3 * P + c0:3 * P + c0 + D_KEY])
                kr = jax.nn.relu(g_ref[rows, c0:c0 + D_KEY])
                vv = g_ref[rows, P + c0:P + c0 + D_KEY]
                ck = kr * aw
                e_aw = jnp.dot(aw.astype(bf16), sel,
                               preferred_element_type=f32)
                e_ae = jnp.dot(ae.astype(bf16), sel,
                               preferred_element_type=f32)
                e_ck = jnp.dot(ck.astype(bf16), sel,
                               preferred_element_type=f32)
                e_v = jnp.dot(vv.astype(bf16), til,
                              preferred_element_type=f32)
                dst_ref[:, h] = (src_ref[:, h]
                                 * (1.0 - e_aw * jnp.maximum(e_ae, md2))
                                 + e_ck * md2 * e_v)

        @pl.when(hp >= 2)
        def _():
            pltpu.make_async_copy(
                obuf.at[hp % 2], om_hbm.at[pl.ds(0, BB)],
                out_sems.at[hp % 2]).wait()

        @pl.when(hp < PRE)
        def _():
            update(mring.at[hp], obuf.at[hp % 2])

        @pl.when(hp >= PRE)
        def _():
            update(m_auto, obuf.at[hp % 2])

        pltpu.make_async_copy(
            obuf.at[hp % 2], om_hbm.at[pl.ds(pl.multiple_of(hp * BB, BB), BB)],
            out_sems.at[hp % 2]).start()

        ndt = ndt_ref[...]
        aw_f = jax.nn.sigmoid(g_ref[rows, 2 * P:3 * P])
        ae_f = jax.nn.sigmoid(g_ref[rows, 3 * P:4 * P])
        kr_f = jax.nn.relu(g_ref[rows, 0:P])
        ck_f = kr_f * aw_f
        n_old = n2_ref[rows, :]
        on_ref[rows, :] = (n_old * (1.0 - aw_f * jnp.maximum(ae_f, ndt))
                           + ck_f * ndt)

        @pl.when(hp == NCH - 1)
        def _():
            for par in range(2):
                pltpu.make_async_copy(
                    obuf.at[par], om_hbm.at[pl.ds(0, BB)],
                    out_sems.at[par]).wait()


@jax.jit
def kernel(tensor, matrix, normalizer, sel_index, sel_probs,
           key_kernel, key_bias, value_kernel, value_bias,
           write_kernel, write_bias, erase_kernel, erase_bias,
           key_decay_logits, value_decay_logits):
    m4 = matrix.reshape(B, H, DKV)
    n2 = normalizer.reshape(B, P)

    nd = jax.nn.sigmoid(key_decay_logits)
    md2 = (jax.nn.sigmoid(value_decay_logits)
           * nd[:, None]).reshape(1, DKV)
    ndt = jnp.tile(nd, H).reshape(1, P)
    eye = jnp.eye(D_KEY, dtype=bf16)
    sel_mat = jnp.repeat(eye, D_VALUE, axis=1)
    tile_mat = jnp.tile(eye, (1, D_KEY))

    w_spec = pl.BlockSpec(
        (1, D_MODEL, DT),
        lambda s: (jnp.where(s < NA, s % BANK, BANK - 1), 0,
                   jnp.minimum(s // BANK, NT - 1)))
    b_spec = pl.BlockSpec((BANK, 1, P), lambda s: (0, 0, 0))
    m_auto_spec = pl.BlockSpec(
        (BB, H, DKV),
        lambda s: (jnp.clip(s - NA, PRE, NCH - 1), 0, 0))

    def full(shape):
        return pl.BlockSpec(shape, lambda s: (0,) * len(shape))

    om, on = pl.pallas_call(
        _body,
        grid=(NA + NCH,),
        in_specs=[full((B, D_MODEL)), full((B, TOPK)), full((B, TOPK)),
                  full((B, P)), full((1, P)), full((1, DKV)),
                  full((D_KEY, DKV)), full((D_KEY, DKV)),
                  w_spec, b_spec, w_spec, b_spec,
                  w_spec, b_spec, w_spec, b_spec,
                  pl.BlockSpec(memory_space=pl.ANY), m_auto_spec],
        out_specs=[pl.BlockSpec(memory_space=pl.ANY), full((B, P))],
        out_shape=[jax.ShapeDtypeStruct((B, H, DKV), f32),
                   jax.ShapeDtypeStruct((B, P), f32)],
        scratch_shapes=[pltpu.VMEM((B, BANK * P), f32),
                        pltpu.VMEM((PRE, BB, H, DKV), f32),
                        pltpu.VMEM((2, BB, H, DKV), f32),
                        pltpu.SemaphoreType.DMA((PRE,)),
                        pltpu.SemaphoreType.DMA((2,))],
        compiler_params=pltpu.CompilerParams(
            dimension_semantics=("arbitrary",),
            vmem_limit_bytes=63 * 1024 * 1024),
    )(tensor, sel_index, sel_probs, n2, ndt, md2, sel_mat, tile_mat,
      key_kernel, key_bias.reshape(BANK, 1, P),
      value_kernel, value_bias.reshape(BANK, 1, P),
      write_kernel, write_bias.reshape(BANK, 1, P),
      erase_kernel, erase_bias.reshape(BANK, 1, P),
      m4, m4)

    return (om.reshape(B, H, D_KEY, D_VALUE), on.reshape(B, H, D_KEY))


# speedup vs baseline: 1.2225x; 1.2225x over previous
import jax
import jax.numpy as jnp
from jax.experimental import pallas as pl
from jax.experimental.pallas import tpu as pltpu

B = 256
D_MODEL = 1024
D_KEY = 64
D_VALUE = 64
DKV = D_KEY * D_VALUE
H = 16
BANK = 4
TOPK = 2
P = H * D_KEY
DT = 256
NT = P // DT
NA = NT * BANK
NPAIR = H // 2
CHW = 2 * DKV
RING = 3
PRE = 2

f32 = jnp.float32
bf16 = jnp.bfloat16


def _body(x_ref, idx_ref, probs_ref, n2_ref, ndt_ref, md2_ref,
          sel_mat_ref, tile_mat_ref,
          wk_ref, bk_ref, wv_ref, bv_ref, ww_ref, bw_ref, we_ref, be_ref,
          m_hbm, om_hbm, on_ref,
          g_ref, mring, in_sems, out_sems):
    s = pl.program_id(0)

    def in_copy(c):
        return pltpu.make_async_copy(
            m_hbm.at[:, c], mring.at[c % RING], in_sems.at[c % RING])

    def out_copy(c):
        return pltpu.make_async_copy(
            mring.at[c % RING], om_hbm.at[:, c], out_sems.at[c % RING])

    @pl.when(s == 0)
    def _():
        for c in range(PRE):
            in_copy(c).start()

    @pl.when(s < NA)
    def _():
        t = s // BANK
        n = s % BANK
        t_off = pl.multiple_of(t * DT, DT)
        sn = jnp.zeros((B, 1), f32)
        for k in range(TOPK):
            sn = sn + jnp.where(idx_ref[:, k:k + 1] == n,
                                probs_ref[:, k:k + 1], 0.0)
        x = x_ref[...].astype(bf16)
        for p, (w_ref, b_ref) in enumerate(
                ((wk_ref, bk_ref), (wv_ref, bv_ref),
                 (ww_ref, bw_ref), (we_ref, be_ref))):
            y = jnp.dot(x, w_ref[0].astype(bf16),
                        preferred_element_type=f32)
            bias = b_ref[n, 0:1, pl.ds(t_off, DT)]
            contrib = sn * (y + bias)
            col = pl.ds(p * P + t_off, DT)

            @pl.when(n == 0)
            def _(col=col, contrib=contrib):
                g_ref[:, col] = contrib

            @pl.when(n != 0)
            def _(col=col, contrib=contrib):
                g_ref[:, col] = g_ref[:, col] + contrib

    @pl.when(s >= NA)
    def _():
        hp = s - NA
        in_copy(hp).wait()
        slot = mring.at[hp % RING]
        goff = pl.multiple_of(hp * 2 * D_KEY, 2 * D_KEY)

        def gpair(p):
            return g_ref[:, pl.ds(p * P + goff, 2 * D_KEY)]

        kr2 = jax.nn.relu(gpair(0))
        vv2 = gpair(1)
        aw2 = jax.nn.sigmoid(gpair(2))
        ae2 = jax.nn.sigmoid(gpair(3))
        ck2 = kr2 * aw2

        sel = sel_mat_ref[...]
        til = tile_mat_ref[...]
        md2 = md2_ref[...]

        for j in (0, 1):
            cs = slice(j * D_KEY, (j + 1) * D_KEY)
            aw = aw2[:, cs]
            ae = ae2[:, cs]
            ck = ck2[:, cs]
            vv = vv2[:, cs]
            e_aw = jnp.dot(aw.astype(bf16), sel, preferred_element_type=f32)
            e_ae = jnp.dot(ae.astype(bf16), sel, preferred_element_type=f32)
            e_ck = jnp.dot(ck.astype(bf16), sel, preferred_element_type=f32)
            e_v = jnp.dot(vv.astype(bf16), til, preferred_element_type=f32)
            mcs = slice(j * DKV, (j + 1) * DKV)
            slot[:, mcs] = (slot[:, mcs]
                            * (1.0 - e_aw * jnp.maximum(e_ae, md2))
                            + e_ck * md2 * e_v)
        out_copy(hp).start()

        nd_row = ndt_ref[0:1, pl.ds(goff, 2 * D_KEY)]
        n_old = n2_ref[:, pl.ds(goff, 2 * D_KEY)]
        on_ref[:, pl.ds(goff, 2 * D_KEY)] = (
            n_old * (1.0 - aw2 * jnp.maximum(ae2, nd_row)) + ck2 * nd_row)

        c = hp + PRE

        @pl.when(jnp.logical_and(c < NPAIR, c - RING >= 0))
        def _():
            out_copy(c - RING).wait()

        @pl.when(c < NPAIR)
        def _():
            in_copy(c).start()

        @pl.when(hp == NPAIR - 1)
        def _():
            for c2 in range(NPAIR - RING, NPAIR):
                out_copy(c2).wait()


@jax.jit
def kernel(tensor, matrix, normalizer, sel_index, sel_probs,
           key_kernel, key_bias, value_kernel, value_bias,
           write_kernel, write_bias, erase_kernel, erase_bias,
           key_decay_logits, value_decay_logits):
    m3 = matrix.reshape(B, NPAIR, CHW)
    n2 = normalizer.reshape(B, P)

    nd = jax.nn.sigmoid(key_decay_logits)
    md2 = (jax.nn.sigmoid(value_decay_logits)
           * nd[:, None]).reshape(1, DKV)
    ndt = jnp.tile(nd, H).reshape(1, P)
    eye = jnp.eye(D_KEY, dtype=bf16)
    sel_mat = jnp.repeat(eye, D_VALUE, axis=1)
    tile_mat = jnp.tile(eye, (1, D_KEY))

    w_spec = pl.BlockSpec(
        (1, D_MODEL, DT),
        lambda s: (jnp.where(s < NA, s % BANK, BANK - 1), 0,
                   jnp.minimum(s // BANK, NT - 1)))
    b_spec = pl.BlockSpec((BANK, 1, P), lambda s: (0, 0, 0))

    def full(shape):
        return pl.BlockSpec(shape, lambda s: (0,) * len(shape))

    om, on = pl.pallas_call(
        _body,
        grid=(NA + NPAIR,),
        in_specs=[full((B, D_MODEL)), full((B, TOPK)), full((B, TOPK)),
                  full((B, P)), full((1, P)), full((1, DKV)),
                  full((D_KEY, DKV)), full((D_KEY, DKV)),
                  w_spec, b_spec, w_spec, b_spec,
                  w_spec, b_spec, w_spec, b_spec,
                  pl.BlockSpec(memory_space=pl.ANY)],
        out_specs=[pl.BlockSpec(memory_space=pl.ANY), full((B, P))],
        out_shape=[jax.ShapeDtypeStruct((B, NPAIR, CHW), f32),
                   jax.ShapeDtypeStruct((B, P), f32)],
        scratch_shapes=[pltpu.VMEM((B, BANK * P), f32),
                        pltpu.VMEM((RING, B, CHW), f32),
                        pltpu.SemaphoreType.DMA((RING,)),
                        pltpu.SemaphoreType.DMA((RING,))],
        compiler_params=pltpu.CompilerParams(
            dimension_semantics=("arbitrary",),
            vmem_limit_bytes=63 * 1024 * 1024),
    )(tensor, sel_index, sel_probs, n2, ndt, md2, sel_mat, tile_mat,
      key_kernel, key_bias.reshape(BANK, 1, P),
      value_kernel, value_bias.reshape(BANK, 1, P),
      write_kernel, write_bias.reshape(BANK, 1, P),
      erase_kernel, erase_bias.reshape(BANK, 1, P),
      m3)

    return (om.reshape(B, H, D_KEY, D_VALUE), on.reshape(B, H, D_KEY))
